# Initial kernel scaffold; baseline (speedup 1.0000x reference)
import jax
import jax.numpy as jnp
from jax.experimental import pallas as pl


def kernel(users, items, user_emb_w, item_emb_w, graph_idx, graph_val):
    raise NotImplementedError("write your pallas kernel here")


# speedup vs baseline: 20.8916x; 20.8916x over previous
import functools

import jax
import jax.numpy as jnp
from jax import lax
from jax.experimental import pallas as pl
from jax.experimental.pallas import tpu as pltpu
from jax.experimental.pallas import tpu_sc as plsc

N_USERS = 50000
N_ITEMS = 50000
N_NODES = N_USERS + N_ITEMS
EMB = 16
NC = 2
NS = 16
NW = NC * NS
CH = 128
STAGE = 16
ROWS_PER_TILE = 6256
NP = ROWS_PER_TILE * NS
ZCH = 368


def _mesh():
    return plsc.VectorSubcoreMesh(core_axis_name="c", subcore_axis_name="s")


def _zero_rows(buf):
    def zr(e, _):
        buf[e, :] = jnp.zeros((EMB,), jnp.float32)
        return 0
    lax.fori_loop(0, ZCH, zr, 0)


@functools.lru_cache(maxsize=None)
def _layer_kernel(first, cpt):
    outs = [jax.ShapeDtypeStruct((2 * NP, EMB), jnp.float32)]
    if not first:
        outs = [jax.ShapeDtypeStruct((2 * NP, EMB), jnp.float32)] + outs
    scratch = [
        pltpu.VMEM_SHARED((NP, EMB), jnp.float32),
        pltpu.VMEM((ZCH, EMB), jnp.float32),
        pltpu.VMEM((ZCH, EMB), jnp.float32),
        pltpu.VMEM((STAGE, CH), jnp.int32),
        pltpu.VMEM((STAGE, CH), jnp.int32),
        pltpu.VMEM((STAGE, CH), jnp.float32),
        pltpu.VMEM((CH, EMB), jnp.float32),
        pltpu.SemaphoreType.DMA,
    ]

    @functools.partial(
        pl.kernel,
        out_type=tuple(outs) if len(outs) > 1 else outs[0],
        scratch_types=scratch,
        mesh=_mesh(),
        compiler_params=pltpu.CompilerParams(use_tc_tiling_on_sc=False, needs_layout_passes=False),
    )
    def k(gin, src_r, dst_r, val_r, *refs):
        if first:
            p_ref, acc, b0, b1, sidx, didx, vval, rows, gsem = refs
            gtab = gin
        else:
            t_ref, p_ref, acc, b0, b1, sidx, didx, vval, rows, gsem = refs
            gtab = t_ref
        c = lax.axis_index("c")
        s = lax.axis_index("s")
        w = c * NS + s
        base = s * ROWS_PER_TILE

        if not first:
            def pa(kk, _):
                r0 = base + kk * ZCH
                pltpu.sync_copy(gin.at[pl.ds(r0, ZCH)], b0)
                pltpu.sync_copy(gin.at[pl.ds(NP + r0, ZCH)], b1)

                def ar(e, _):
                    b0[e, :] = b0[e, :] + b1[e, :]
                    return 0

                lax.fori_loop(0, ZCH, ar, 0)
                pltpu.sync_copy(b0, t_ref.at[pl.ds(c * NP + r0, ZCH)])
                return 0

            lax.fori_loop(0, ROWS_PER_TILE // ZCH, pa, 0)

        _zero_rows(b0)

        def zc(kk, _):
            pltpu.sync_copy(b0, acc.at[pl.ds(base + kk * ZCH, ZCH)])
            return 0

        lax.fori_loop(0, ROWS_PER_TILE // ZCH, zc, 0)
        plsc.subcore_barrier()

        def stage_it(t_, _):
            j0 = t_ * STAGE
            pltpu.sync_copy(src_r.at[w, pl.ds(j0, STAGE)], sidx)
            pltpu.sync_copy(dst_r.at[w, pl.ds(j0, STAGE)], didx)
            pltpu.sync_copy(val_r.at[w, pl.ds(j0, STAGE)], vval)

            def chunk_it(j, _):
                pltpu.async_copy(gtab.at[sidx.at[j]], rows, gsem).wait()
                for g in range(CH // 16):
                    vv = vval[j, pl.ds(g * 16, 16)]
                    for e in range(16):
                        r = g * 16 + e
                        rows[r, :] = rows[r, :] * vv[e]
                pltpu.sync_copy(rows, acc.at[didx.at[j]], add=True)
                return 0

            lax.fori_loop(0, STAGE, chunk_it, 0)
            return 0

        lax.fori_loop(0, cpt // STAGE, stage_it, 0)
        plsc.subcore_barrier()

        pltpu.sync_copy(
            acc.at[pl.ds(base, ROWS_PER_TILE)],
            p_ref.at[pl.ds(c * NP + base, ROWS_PER_TILE)],
        )

    return k


@functools.lru_cache(maxsize=None)
def _score_kernel(batch):
    bpt = batch // NW

    scratch = [
        pltpu.VMEM((bpt,), jnp.int32),
        pltpu.VMEM((bpt,), jnp.int32),
        pltpu.VMEM((bpt,), jnp.int32),
        pltpu.VMEM((bpt,), jnp.int32),
        pltpu.VMEM((bpt, EMB), jnp.float32),
        pltpu.VMEM((bpt, EMB), jnp.float32),
        pltpu.VMEM((bpt, EMB), jnp.float32),
        pltpu.VMEM((bpt,), jnp.float32),
        pltpu.SemaphoreType.DMA,
    ]

    @functools.partial(
        pl.kernel,
        out_type=jax.ShapeDtypeStruct((batch,), jnp.float32),
        scratch_types=scratch,
        mesh=_mesh(),
        compiler_params=pltpu.CompilerParams(use_tc_tiling_on_sc=False, needs_layout_passes=False),
    )
    def k(users, items, e0, t1, t2, p3, out,
          uidx, iidx, uhi, ihi, usum, isum, rbuf, outv, gsem):
        c = lax.axis_index("c")
        s = lax.axis_index("s")
        w = c * NS + s
        b0 = w * bpt
        pltpu.sync_copy(users.at[pl.ds(b0, bpt)], uidx)
        pltpu.sync_copy(items.at[pl.ds(b0, bpt)], iidx)
        nvec = bpt // 16
        for v in range(nvec):
            sl = pl.ds(v * 16, 16)
            iidx[sl] = iidx[sl] + N_USERS
        for v in range(nvec):
            sl = pl.ds(v * 16, 16)
            uhi[sl] = uidx[sl] + NP
        for v in range(nvec):
            sl = pl.ds(v * 16, 16)
            ihi[sl] = iidx[sl] + NP

        def accum(tab, idx, sumbuf, is_first):
            if is_first:
                pltpu.async_copy(tab.at[idx], sumbuf, gsem).wait()
                return
            pltpu.async_copy(tab.at[idx], rbuf, gsem).wait()

            def add_row(e, _):
                sumbuf[e, :] = sumbuf[e, :] + rbuf[e, :]
                return 0

            lax.fori_loop(0, bpt, add_row, 0)

        accum(e0, uidx, usum, True)
        accum(t1, uidx, usum, False)
        accum(t2, uidx, usum, False)
        accum(p3, uidx, usum, False)
        accum(p3, uhi, usum, False)

        accum(e0, iidx, isum, True)
        accum(t1, iidx, isum, False)
        accum(t2, iidx, isum, False)
        accum(p3, iidx, isum, False)
        accum(p3, ihi, isum, False)

        lanes = lax.iota(jnp.int32, 16)

        def dot_g(g, _):
            accv = jnp.zeros((16,), jnp.float32)
            for e in range(16):
                row = g * 16 + e
                sv = jnp.sum(usum[row, :] * isum[row, :]) * jnp.float32(0.0625)
                accv = jnp.where(lanes == e, sv, accv)
            outv[pl.ds(g * 16, 16)] = accv
            return 0

        lax.fori_loop(0, bpt // 16, dot_g, 0)
        pltpu.sync_copy(outv, out.at[pl.ds(b0, bpt)])

    return k


def kernel(users, items, user_emb_w, item_emb_w, graph_idx, graph_val):
    e0 = jnp.concatenate([user_emb_w, item_emb_w], axis=0)
    zpad = jnp.zeros((NP - N_NODES, EMB), jnp.float32)
    e0dup = jnp.concatenate([e0, zpad, e0, zpad], axis=0)
    nnz = graph_val.shape[0]
    cpt = -(-nnz // (NW * CH))
    cpt = -(-cpt // STAGE) * STAGE
    pad = NW * cpt * CH - nnz
    src = jnp.pad(graph_idx[1], (0, pad)).reshape(NW, cpt, CH)
    dst = jnp.pad(graph_idx[0], (0, pad)).reshape(NW, cpt, CH)
    val = jnp.pad(graph_val, (0, pad)).reshape(NW, cpt, CH)
    src = src + (jnp.arange(NW, dtype=jnp.int32) // NS)[:, None, None] * NP

    p1 = _layer_kernel(True, cpt)(e0dup, src, dst, val)
    t1, p2 = _layer_kernel(False, cpt)(p1, src, dst, val)
    t2, p3 = _layer_kernel(False, cpt)(p2, src, dst, val)
    scores = _score_kernel(int(users.shape[0]))(users, items, e0, t1, t2, p3)
    return scores


# speedup vs baseline: 35.9694x; 1.7217x over previous
import functools

import jax
import jax.numpy as jnp
from jax import lax
from jax.experimental import pallas as pl
from jax.experimental.pallas import tpu as pltpu
from jax.experimental.pallas import tpu_sc as plsc

N_USERS = 50000
N_ITEMS = 50000
N_NODES = N_USERS + N_ITEMS
EMB = 16
NC = 2
NS = 16
NW = NC * NS
CH = 128
SSTAGE = 14
ROWS_PER_TILE = 6256
NP = ROWS_PER_TILE * NS
ZCH = 184


def _mesh():
    return plsc.VectorSubcoreMesh(core_axis_name="c", subcore_axis_name="s")


def _zero_rows(buf):
    def zr(e, _):
        buf[e, :] = jnp.zeros((EMB,), jnp.float32)
        return 0
    lax.fori_loop(0, ZCH, zr, 0)


@functools.lru_cache(maxsize=None)
def _layer_kernel(first, cpt):
    outs = [jax.ShapeDtypeStruct((2 * NP, EMB), jnp.float32)]
    if not first:
        outs = [jax.ShapeDtypeStruct((2 * NP, EMB), jnp.float32)] + outs
    scratch = [
        pltpu.VMEM_SHARED((NP, EMB), jnp.float32),
        pltpu.VMEM((ZCH, EMB), jnp.float32),
        pltpu.VMEM((ZCH, EMB), jnp.float32),
        pltpu.VMEM((2, SSTAGE, CH), jnp.int32),
        pltpu.VMEM((2, SSTAGE, CH), jnp.int32),
        pltpu.VMEM((2, SSTAGE, CH), jnp.float32),
        pltpu.VMEM((CH, EMB), jnp.float32),
        pltpu.VMEM((CH, EMB), jnp.float32),
        pltpu.SemaphoreType.DMA,
        pltpu.SemaphoreType.DMA,
        pltpu.SemaphoreType.DMA,
        pltpu.SemaphoreType.DMA,
    ]

    @functools.partial(
        pl.kernel,
        out_type=tuple(outs) if len(outs) > 1 else outs[0],
        scratch_types=scratch,
        mesh=_mesh(),
        compiler_params=pltpu.CompilerParams(use_tc_tiling_on_sc=False, needs_layout_passes=False),
    )
    def k(gin, src_r, dst_r, val_r, *refs):
        if first:
            (p_ref, acc, b0, b1, sidx, didx, vval, rows0, rows1,
             gsem0, gsem1, stg0, stg1) = refs
            gtab = gin
        else:
            (t_ref, p_ref, acc, b0, b1, sidx, didx, vval, rows0, rows1,
             gsem0, gsem1, stg0, stg1) = refs
            gtab = t_ref
        c = lax.axis_index("c")
        s = lax.axis_index("s")
        w = c * NS + s
        base = s * ROWS_PER_TILE

        if not first:
            def pa(kk, _):
                r0 = base + kk * ZCH
                pltpu.sync_copy(gin.at[pl.ds(r0, ZCH)], b0)
                pltpu.sync_copy(gin.at[pl.ds(NP + r0, ZCH)], b1)

                def ar(e, _):
                    b0[e, :] = b0[e, :] + b1[e, :]
                    return 0

                lax.fori_loop(0, ZCH, ar, 0)
                pltpu.sync_copy(b0, t_ref.at[pl.ds(c * NP + r0, ZCH)])
                return 0

            lax.fori_loop(0, ROWS_PER_TILE // ZCH, pa, 0)

        _zero_rows(b0)

        def zc(kk, _):
            pltpu.sync_copy(b0, acc.at[pl.ds(base + kk * ZCH, ZCH)])
            return 0

        lax.fori_loop(0, ROWS_PER_TILE // ZCH, zc, 0)
        plsc.subcore_barrier()

        def stage_in(t_, slot, sem):
            j0 = t_ * SSTAGE
            pltpu.async_copy(src_r.at[w, pl.ds(j0, SSTAGE)], sidx.at[slot], sem)
            pltpu.async_copy(dst_r.at[w, pl.ds(j0, SSTAGE)], didx.at[slot], sem)
            pltpu.async_copy(val_r.at[w, pl.ds(j0, SSTAGE)], vval.at[slot], sem)

        def stage_wait(slot, sem):
            for ref, dref in ((src_r, sidx), (dst_r, didx), (val_r, vval)):
                pltpu.make_async_copy(
                    ref.at[w, pl.ds(0, SSTAGE)], dref.at[slot], sem
                ).wait()

        def scale(rbuf, vs, j):
            for g in range(CH // 16):
                vv = vs[j, pl.ds(g * 16, 16)]
                for e in range(16):
                    r = g * 16 + e
                    rbuf[r, :] = rbuf[r, :] * vv[e]

        def gather(rbuf, ss, j, sem):
            pltpu.async_copy(gtab.at[ss.at[j]], rbuf, sem)

        def gwait(rbuf, sem):
            pltpu.make_async_copy(gtab.at[pl.ds(0, CH)], rbuf, sem).wait()

        def process_stage(ss, ds, vs, last):
            gather(rows0, ss, 0, gsem0)

            def pair(i, _):
                a = 2 * i
                gather(rows1, ss, a + 1, gsem1)
                gwait(rows0, gsem0)
                scale(rows0, vs, a)
                pltpu.sync_copy(rows0, acc.at[ds.at[a]], add=True)

                @pl.when(i < SSTAGE // 2 - 1)
                def _():
                    gather(rows0, ss, a + 2, gsem0)

                gwait(rows1, gsem1)
                scale(rows1, vs, a + 1)
                pltpu.sync_copy(rows1, acc.at[ds.at[a + 1]], add=True)
                return 0

            lax.fori_loop(0, SSTAGE // 2, pair, 0)

        nstg = cpt // SSTAGE
        stage_in(0, 0, stg0)
        stage_in(1, 1, stg1)

        def outer(u, _):
            t0 = 2 * u
            stage_wait(0, stg0)
            process_stage(sidx.at[0], didx.at[0], vval.at[0], False)

            @pl.when(u < nstg // 2 - 1)
            def _():
                stage_in(t0 + 2, 0, stg0)

            stage_wait(1, stg1)
            process_stage(sidx.at[1], didx.at[1], vval.at[1], False)

            @pl.when(u < nstg // 2 - 1)
            def _():
                stage_in(t0 + 3, 1, stg1)

            return 0

        lax.fori_loop(0, nstg // 2, outer, 0)
        plsc.subcore_barrier()

        pltpu.sync_copy(
            acc.at[pl.ds(base, ROWS_PER_TILE)],
            p_ref.at[pl.ds(c * NP + base, ROWS_PER_TILE)],
        )

    return k


@functools.lru_cache(maxsize=None)
def _score_kernel(batch):
    bpt = batch // NW

    scratch = [
        pltpu.VMEM((bpt,), jnp.int32),
        pltpu.VMEM((bpt,), jnp.int32),
        pltpu.VMEM((bpt,), jnp.int32),
        pltpu.VMEM((bpt,), jnp.int32),
        pltpu.VMEM((bpt, EMB), jnp.float32),
        pltpu.VMEM((bpt, EMB), jnp.float32),
        pltpu.VMEM((bpt, EMB), jnp.float32),
        pltpu.VMEM((bpt,), jnp.float32),
        pltpu.SemaphoreType.DMA,
    ]

    @functools.partial(
        pl.kernel,
        out_type=jax.ShapeDtypeStruct((batch,), jnp.float32),
        scratch_types=scratch,
        mesh=_mesh(),
        compiler_params=pltpu.CompilerParams(use_tc_tiling_on_sc=False, needs_layout_passes=False),
    )
    def k(users, items, e0, t1, t2, p3, out,
          uidx, iidx, uhi, ihi, usum, isum, rbuf, outv, gsem):
        c = lax.axis_index("c")
        s = lax.axis_index("s")
        w = c * NS + s
        b0 = w * bpt
        pltpu.sync_copy(users.at[pl.ds(b0, bpt)], uidx)
        pltpu.sync_copy(items.at[pl.ds(b0, bpt)], iidx)
        nvec = bpt // 16
        for v in range(nvec):
            sl = pl.ds(v * 16, 16)
            iidx[sl] = iidx[sl] + N_USERS
        for v in range(nvec):
            sl = pl.ds(v * 16, 16)
            uhi[sl] = uidx[sl] + NP
        for v in range(nvec):
            sl = pl.ds(v * 16, 16)
            ihi[sl] = iidx[sl] + NP

        def accum(tab, idx, sumbuf, is_first):
            if is_first:
                pltpu.async_copy(tab.at[idx], sumbuf, gsem).wait()
                return
            pltpu.async_copy(tab.at[idx], rbuf, gsem).wait()

            def add_row(e, _):
                sumbuf[e, :] = sumbuf[e, :] + rbuf[e, :]
                return 0

            lax.fori_loop(0, bpt, add_row, 0)

        accum(e0, uidx, usum, True)
        accum(t1, uidx, usum, False)
        accum(t2, uidx, usum, False)
        accum(p3, uidx, usum, False)
        accum(p3, uhi, usum, False)

        accum(e0, iidx, isum, True)
        accum(t1, iidx, isum, False)
        accum(t2, iidx, isum, False)
        accum(p3, iidx, isum, False)
        accum(p3, ihi, isum, False)

        lanes = lax.iota(jnp.int32, 16)

        def dot_g(g, _):
            accv = jnp.zeros((16,), jnp.float32)
            for e in range(16):
                row = g * 16 + e
                sv = jnp.sum(usum[row, :] * isum[row, :]) * jnp.float32(0.0625)
                accv = jnp.where(lanes == e, sv, accv)
            outv[pl.ds(g * 16, 16)] = accv
            return 0

        lax.fori_loop(0, bpt // 16, dot_g, 0)
        pltpu.sync_copy(outv, out.at[pl.ds(b0, bpt)])

    return k


def kernel(users, items, user_emb_w, item_emb_w, graph_idx, graph_val):
    e0 = jnp.concatenate([user_emb_w, item_emb_w], axis=0)
    zpad = jnp.zeros((NP - N_NODES, EMB), jnp.float32)
    e0dup = jnp.concatenate([e0, zpad, e0, zpad], axis=0)
    nnz = graph_val.shape[0]
    cpt = -(-nnz // (NW * CH))
    cpt = -(-cpt // SSTAGE) * SSTAGE
    pad = NW * cpt * CH - nnz
    src = jnp.pad(graph_idx[1], (0, pad)).reshape(NW, cpt, CH)
    dst = jnp.pad(graph_idx[0], (0, pad)).reshape(NW, cpt, CH)
    val = jnp.pad(graph_val, (0, pad)).reshape(NW, cpt, CH)
    src = src + (jnp.arange(NW, dtype=jnp.int32) // NS)[:, None, None] * NP

    p1 = _layer_kernel(True, cpt)(e0dup, src, dst, val)
    t1, p2 = _layer_kernel(False, cpt)(p1, src, dst, val)
    t2, p3 = _layer_kernel(False, cpt)(p2, src, dst, val)
    scores = _score_kernel(int(users.shape[0]))(users, items, e0, t1, t2, p3)
    return scores


# speedup vs baseline: 41.0632x; 1.1416x over previous
import functools

import jax
import jax.numpy as jnp
import numpy as np
from jax import lax
from jax.experimental import pallas as pl
from jax.experimental.pallas import tpu as pltpu
from jax.experimental.pallas import tpu_sc as plsc

N_USERS = 50000
N_ITEMS = 50000
N_NODES = N_USERS + N_ITEMS
EMB = 16
NC = 2
NS = 16
NW = NC * NS
CH = 128
SSTAGE = 14
ROWS_PER_TILE = 6256
NP = ROWS_PER_TILE * NS
ZCH = 184


def _mesh():
    return plsc.VectorSubcoreMesh(core_axis_name="c", subcore_axis_name="s")


def _zero_rows(buf):
    z = jnp.zeros((EMB,), jnp.float32)
    for e in range(ZCH):
        buf[e, :] = z


@functools.lru_cache(maxsize=None)
def _layer_kernel(cpt):
    scratch = [
        pltpu.VMEM_SHARED((NP, EMB), jnp.float32),
        pltpu.VMEM((ZCH, EMB), jnp.float32),
        pltpu.VMEM((2, SSTAGE, CH), jnp.int32),
        pltpu.VMEM((2, SSTAGE, CH), jnp.int32),
        pltpu.VMEM((2, SSTAGE, CH), jnp.float32),
        pltpu.VMEM((CH, EMB), jnp.float32),
        pltpu.VMEM((CH, EMB), jnp.float32),
        pltpu.SemaphoreType.DMA,
        pltpu.SemaphoreType.DMA,
        pltpu.SemaphoreType.DMA,
        pltpu.SemaphoreType.DMA,
    ]

    @functools.partial(
        pl.kernel,
        out_type=jax.ShapeDtypeStruct((2 * NP, EMB), jnp.float32),
        scratch_types=scratch,
        mesh=_mesh(),
        compiler_params=pltpu.CompilerParams(use_tc_tiling_on_sc=False, needs_layout_passes=False),
    )
    def k(gin, src_r, dst_r, val_r, *refs):
        (p_ref, acc, b0, sidx, didx, vval, rows0, rows1,
         gsem0, gsem1, stg0, stg1) = refs
        gtab = gin
        c = lax.axis_index("c")
        s = lax.axis_index("s")
        w = c * NS + s
        base = s * ROWS_PER_TILE

        _zero_rows(b0)

        def zc(kk, _):
            pltpu.sync_copy(b0, acc.at[pl.ds(base + kk * ZCH, ZCH)])
            return 0

        lax.fori_loop(0, ROWS_PER_TILE // ZCH, zc, 0)
        plsc.subcore_barrier()

        def stage_in(t_, slot, sem):
            j0 = t_ * SSTAGE
            pltpu.async_copy(src_r.at[w, pl.ds(j0, SSTAGE)], sidx.at[slot], sem)
            pltpu.async_copy(dst_r.at[w, pl.ds(j0, SSTAGE)], didx.at[slot], sem)
            pltpu.async_copy(val_r.at[w, pl.ds(j0, SSTAGE)], vval.at[slot], sem)

        def stage_wait(slot, sem):
            for ref, dref in ((src_r, sidx), (dst_r, didx), (val_r, vval)):
                pltpu.make_async_copy(
                    ref.at[w, pl.ds(0, SSTAGE)], dref.at[slot], sem
                ).wait()

        def scale(rbuf, vs, j):
            for g in range(CH // 16):
                vv = vs[j, pl.ds(g * 16, 16)]
                for e in range(16):
                    r = g * 16 + e
                    lane = jnp.full((16, 1), e, jnp.int32)
                    bc = lax.gather(
                        vv, lane,
                        lax.GatherDimensionNumbers(
                            offset_dims=(), collapsed_slice_dims=(0,),
                            start_index_map=(0,)),
                        (1,), mode=lax.GatherScatterMode.PROMISE_IN_BOUNDS)
                    rbuf[r, :] = rbuf[r, :] * bc

        def gather(rbuf, ss, j, sem):
            pltpu.async_copy(gtab.at[ss.at[j]], rbuf, sem)

        def gwait(rbuf, sem):
            pltpu.make_async_copy(gtab.at[pl.ds(0, CH)], rbuf, sem).wait()

        def process_stage(ss, ds, vs, last):
            gather(rows0, ss, 0, gsem0)

            def pair(i, _):
                a = 2 * i
                gather(rows1, ss, a + 1, gsem1)
                gwait(rows0, gsem0)
                scale(rows0, vs, a)
                pltpu.sync_copy(rows0, acc.at[ds.at[a]], add=True)

                @pl.when(i < SSTAGE // 2 - 1)
                def _():
                    gather(rows0, ss, a + 2, gsem0)

                gwait(rows1, gsem1)
                scale(rows1, vs, a + 1)
                pltpu.sync_copy(rows1, acc.at[ds.at[a + 1]], add=True)
                return 0

            lax.fori_loop(0, SSTAGE // 2, pair, 0)

        nstg = cpt // SSTAGE
        stage_in(0, 0, stg0)
        stage_in(1, 1, stg1)

        def outer(u, _):
            t0 = 2 * u
            stage_wait(0, stg0)
            process_stage(sidx.at[0], didx.at[0], vval.at[0], False)

            @pl.when(u < nstg // 2 - 1)
            def _():
                stage_in(t0 + 2, 0, stg0)

            stage_wait(1, stg1)
            process_stage(sidx.at[1], didx.at[1], vval.at[1], False)

            @pl.when(u < nstg // 2 - 1)
            def _():
                stage_in(t0 + 3, 1, stg1)

            return 0

        lax.fori_loop(0, nstg // 2, outer, 0)
        plsc.subcore_barrier()

        pltpu.sync_copy(
            acc.at[pl.ds(base, ROWS_PER_TILE)],
            p_ref.at[pl.ds(c * NP + base, ROWS_PER_TILE)],
        )

    return k


_TCR = NP * EMB // 128


@functools.lru_cache(maxsize=None)
def _tc_pair_sum():
    def body(a_ref, b_ref, o_ref):
        o_ref[...] = a_ref[...] + b_ref[...]

    blk = _TCR // 4
    return pl.pallas_call(
        body,
        grid=(4,),
        in_specs=[
            pl.BlockSpec((None, blk, 128), lambda i: (0, i, 0)),
            pl.BlockSpec((None, blk, 128), lambda i: (1, i, 0)),
        ],
        out_specs=pl.BlockSpec((blk, 128), lambda i: (i, 0)),
        out_shape=jax.ShapeDtypeStruct((_TCR, 128), jnp.float32),
    )


@functools.lru_cache(maxsize=None)
def _score_kernel(batch):
    bpt = batch // NW

    scratch = [
        pltpu.VMEM((bpt,), jnp.int32),
        pltpu.VMEM((bpt,), jnp.int32),
        pltpu.VMEM((bpt,), jnp.int32),
        pltpu.VMEM((bpt,), jnp.int32),
        pltpu.VMEM((bpt, EMB), jnp.float32),
        pltpu.VMEM((bpt, EMB), jnp.float32),
        pltpu.VMEM((bpt, EMB), jnp.float32),
        pltpu.VMEM((bpt,), jnp.float32),
        pltpu.SemaphoreType.DMA,
    ]

    @functools.partial(
        pl.kernel,
        out_type=jax.ShapeDtypeStruct((batch,), jnp.float32),
        scratch_types=scratch,
        mesh=_mesh(),
        compiler_params=pltpu.CompilerParams(use_tc_tiling_on_sc=False, needs_layout_passes=False),
    )
    def k(users, items, e0, t1, t2, p3, out,
          uidx, iidx, uhi, ihi, usum, isum, rbuf, outv, gsem):
        c = lax.axis_index("c")
        s = lax.axis_index("s")
        w = c * NS + s
        b0 = w * bpt
        pltpu.sync_copy(users.at[pl.ds(b0, bpt)], uidx)
        pltpu.sync_copy(items.at[pl.ds(b0, bpt)], iidx)
        nvec = bpt // 16
        for v in range(nvec):
            sl = pl.ds(v * 16, 16)
            iidx[sl] = iidx[sl] + N_USERS
        for v in range(nvec):
            sl = pl.ds(v * 16, 16)
            uhi[sl] = uidx[sl] + NP
        for v in range(nvec):
            sl = pl.ds(v * 16, 16)
            ihi[sl] = iidx[sl] + NP

        def accum(tab, idx, sumbuf, is_first):
            if is_first:
                pltpu.async_copy(tab.at[idx], sumbuf, gsem).wait()
                return
            pltpu.async_copy(tab.at[idx], rbuf, gsem).wait()

            for e in range(bpt):
                sumbuf[e, :] = sumbuf[e, :] + rbuf[e, :]

        accum(e0, uidx, usum, True)
        accum(t1, uidx, usum, False)
        accum(t2, uidx, usum, False)
        accum(p3, uidx, usum, False)
        accum(p3, uhi, usum, False)

        accum(e0, iidx, isum, True)
        accum(t1, iidx, isum, False)
        accum(t2, iidx, isum, False)
        accum(p3, iidx, isum, False)
        accum(p3, ihi, isum, False)

        lanes = lax.iota(jnp.int32, 16)

        def dot_g(g, _):
            accv = jnp.zeros((16,), jnp.float32)
            for e in range(16):
                row = g * 16 + e
                sv = jnp.sum(usum[row, :] * isum[row, :]) * jnp.float32(0.0625)
                accv = jnp.where(lanes == e, sv, accv)
            outv[pl.ds(g * 16, 16)] = accv
            return 0

        lax.fori_loop(0, bpt // 16, dot_g, 0)
        pltpu.sync_copy(outv, out.at[pl.ds(b0, bpt)])

    return k


def kernel(users, items, user_emb_w, item_emb_w, graph_idx, graph_val):
    e0 = jnp.concatenate([user_emb_w, item_emb_w], axis=0)
    zpad = jnp.zeros((NP - N_NODES, EMB), jnp.float32)
    e0p = jnp.concatenate([e0, zpad], axis=0)
    nnz = graph_val.shape[0]
    cpt = -(-nnz // (NW * CH))
    cpt = -(-cpt // SSTAGE) * SSTAGE
    pad = NW * cpt * CH - nnz
    src = jnp.pad(graph_idx[1], (0, pad)).reshape(NW, cpt, CH)
    dst = jnp.pad(graph_idx[0], (0, pad)).reshape(NW, cpt, CH)
    val = jnp.pad(graph_val, (0, pad)).reshape(NW, cpt, CH)

    layer = _layer_kernel(cpt)
    tc_sum = _tc_pair_sum()

    def pair_sum(p):
        pr = p.reshape(2, _TCR, 128)
        return tc_sum(pr, pr).reshape(NP, EMB)

    p1 = layer(e0p, src, dst, val)
    t1 = pair_sum(p1)
    p2 = layer(t1, src, dst, val)
    t2 = pair_sum(p2)
    p3 = layer(t2, src, dst, val)
    scores = _score_kernel(int(users.shape[0]))(users, items, e0p, t1, t2, p3)
    return scores


# speedup vs baseline: 42.0679x; 1.0245x over previous
import functools

import jax
import jax.numpy as jnp
import numpy as np
from jax import lax
from jax.experimental import pallas as pl
from jax.experimental.pallas import tpu as pltpu
from jax.experimental.pallas import tpu_sc as plsc

N_USERS = 50000
N_ITEMS = 50000
N_NODES = N_USERS + N_ITEMS
EMB = 16
NC = 2
NS = 16
NW = NC * NS
CH = 128
SSTAGE = 28
ROWS_PER_TILE = 6256
NP = ROWS_PER_TILE * NS
ZCH = 184


def _mesh():
    return plsc.VectorSubcoreMesh(core_axis_name="c", subcore_axis_name="s")


def _zero_rows(buf):
    z = jnp.zeros((EMB,), jnp.float32)
    for e in range(ZCH):
        buf[e, :] = z


@functools.lru_cache(maxsize=None)
def _layer_kernel(cpt):
    scratch = [
        pltpu.VMEM_SHARED((NP, EMB), jnp.float32),
        pltpu.VMEM((ZCH, EMB), jnp.float32),
        pltpu.VMEM((2, SSTAGE, CH), jnp.int32),
        pltpu.VMEM((2, SSTAGE, CH), jnp.int32),
        pltpu.VMEM((2, SSTAGE, CH), jnp.float32),
        pltpu.VMEM((CH, EMB), jnp.float32),
        pltpu.VMEM((CH, EMB), jnp.float32),
        pltpu.SemaphoreType.DMA,
        pltpu.SemaphoreType.DMA,
        pltpu.SemaphoreType.DMA,
        pltpu.SemaphoreType.DMA,
    ]

    @functools.partial(
        pl.kernel,
        out_type=jax.ShapeDtypeStruct((2 * NP, EMB), jnp.float32),
        scratch_types=scratch,
        mesh=_mesh(),
        compiler_params=pltpu.CompilerParams(use_tc_tiling_on_sc=False, needs_layout_passes=False),
    )
    def k(gin, src_r, dst_r, val_r, *refs):
        (p_ref, acc, b0, sidx, didx, vval, rows0, rows1,
         gsem0, gsem1, stg0, stg1) = refs
        gtab = gin
        c = lax.axis_index("c")
        s = lax.axis_index("s")
        w = c * NS + s
        base = s * ROWS_PER_TILE

        _zero_rows(b0)

        def zc(kk, _):
            pltpu.sync_copy(b0, acc.at[pl.ds(base + kk * ZCH, ZCH)])
            return 0

        lax.fori_loop(0, ROWS_PER_TILE // ZCH, zc, 0)
        plsc.subcore_barrier()

        def stage_in(t_, slot, sem):
            j0 = t_ * SSTAGE
            pltpu.async_copy(src_r.at[w, pl.ds(j0, SSTAGE)], sidx.at[slot], sem)
            pltpu.async_copy(dst_r.at[w, pl.ds(j0, SSTAGE)], didx.at[slot], sem)
            pltpu.async_copy(val_r.at[w, pl.ds(j0, SSTAGE)], vval.at[slot], sem)

        def stage_wait(slot, sem):
            for ref, dref in ((src_r, sidx), (dst_r, didx), (val_r, vval)):
                pltpu.make_async_copy(
                    ref.at[w, pl.ds(0, SSTAGE)], dref.at[slot], sem
                ).wait()

        def scale(rbuf, vs, j):
            for g in range(CH // 16):
                vv = vs[j, pl.ds(g * 16, 16)]
                for e in range(16):
                    r = g * 16 + e
                    lane = jnp.full((16, 1), e, jnp.int32)
                    bc = lax.gather(
                        vv, lane,
                        lax.GatherDimensionNumbers(
                            offset_dims=(), collapsed_slice_dims=(0,),
                            start_index_map=(0,)),
                        (1,), mode=lax.GatherScatterMode.PROMISE_IN_BOUNDS)
                    rbuf[r, :] = rbuf[r, :] * bc

        def gather(rbuf, ss, j, sem):
            pltpu.async_copy(gtab.at[ss.at[j]], rbuf, sem)

        def gwait(rbuf, sem):
            pltpu.make_async_copy(gtab.at[pl.ds(0, CH)], rbuf, sem).wait()

        def process_stage(ss, ds, vs, last):
            gather(rows0, ss, 0, gsem0)

            def pair(i, _):
                a = 2 * i
                gather(rows1, ss, a + 1, gsem1)
                gwait(rows0, gsem0)
                scale(rows0, vs, a)
                pltpu.sync_copy(rows0, acc.at[ds.at[a]], add=True)

                @pl.when(i < SSTAGE // 2 - 1)
                def _():
                    gather(rows0, ss, a + 2, gsem0)

                gwait(rows1, gsem1)
                scale(rows1, vs, a + 1)
                pltpu.sync_copy(rows1, acc.at[ds.at[a + 1]], add=True)
                return 0

            lax.fori_loop(0, SSTAGE // 2, pair, 0)

        nstg = cpt // SSTAGE
        stage_in(0, 0, stg0)
        stage_in(1, 1, stg1)

        def outer(u, _):
            t0 = 2 * u
            stage_wait(0, stg0)
            process_stage(sidx.at[0], didx.at[0], vval.at[0], False)

            @pl.when(u < nstg // 2 - 1)
            def _():
                stage_in(t0 + 2, 0, stg0)

            stage_wait(1, stg1)
            process_stage(sidx.at[1], didx.at[1], vval.at[1], False)

            @pl.when(u < nstg // 2 - 1)
            def _():
                stage_in(t0 + 3, 1, stg1)

            return 0

        lax.fori_loop(0, nstg // 2, outer, 0)
        plsc.subcore_barrier()

        pltpu.sync_copy(
            acc.at[pl.ds(base, ROWS_PER_TILE)],
            p_ref.at[pl.ds(c * NP + base, ROWS_PER_TILE)],
        )

    return k


_TCR = NP * EMB // 128


@functools.lru_cache(maxsize=None)
def _tc_pair_sum():
    def body(a_ref, b_ref, o_ref):
        o_ref[...] = a_ref[...] + b_ref[...]

    blk = _TCR // 4
    return pl.pallas_call(
        body,
        grid=(4,),
        in_specs=[
            pl.BlockSpec((None, blk, 128), lambda i: (0, i, 0)),
            pl.BlockSpec((None, blk, 128), lambda i: (1, i, 0)),
        ],
        out_specs=pl.BlockSpec((blk, 128), lambda i: (i, 0)),
        out_shape=jax.ShapeDtypeStruct((_TCR, 128), jnp.float32),
    )


@functools.lru_cache(maxsize=None)
def _score_kernel(batch):
    bpt = batch // NW

    scratch = [
        pltpu.VMEM((bpt,), jnp.int32),
        pltpu.VMEM((bpt,), jnp.int32),
        pltpu.VMEM((bpt,), jnp.int32),
        pltpu.VMEM((bpt,), jnp.int32),
        pltpu.VMEM((bpt, EMB), jnp.float32),
        pltpu.VMEM((bpt, EMB), jnp.float32),
        pltpu.VMEM((bpt, EMB), jnp.float32),
        pltpu.VMEM((bpt,), jnp.float32),
        pltpu.SemaphoreType.DMA,
    ]

    @functools.partial(
        pl.kernel,
        out_type=jax.ShapeDtypeStruct((batch,), jnp.float32),
        scratch_types=scratch,
        mesh=_mesh(),
        compiler_params=pltpu.CompilerParams(use_tc_tiling_on_sc=False, needs_layout_passes=False),
    )
    def k(users, items, e0, t1, t2, p3, out,
          uidx, iidx, uhi, ihi, usum, isum, rbuf, outv, gsem):
        c = lax.axis_index("c")
        s = lax.axis_index("s")
        w = c * NS + s
        b0 = w * bpt
        pltpu.sync_copy(users.at[pl.ds(b0, bpt)], uidx)
        pltpu.sync_copy(items.at[pl.ds(b0, bpt)], iidx)
        nvec = bpt // 16
        for v in range(nvec):
            sl = pl.ds(v * 16, 16)
            iidx[sl] = iidx[sl] + N_USERS
        for v in range(nvec):
            sl = pl.ds(v * 16, 16)
            uhi[sl] = uidx[sl] + NP
        for v in range(nvec):
            sl = pl.ds(v * 16, 16)
            ihi[sl] = iidx[sl] + NP

        def accum(tab, idx, sumbuf, is_first):
            if is_first:
                pltpu.async_copy(tab.at[idx], sumbuf, gsem).wait()
                return
            pltpu.async_copy(tab.at[idx], rbuf, gsem).wait()

            for e in range(bpt):
                sumbuf[e, :] = sumbuf[e, :] + rbuf[e, :]

        accum(e0, uidx, usum, True)
        accum(t1, uidx, usum, False)
        accum(t2, uidx, usum, False)
        accum(p3, uidx, usum, False)
        accum(p3, uhi, usum, False)

        accum(e0, iidx, isum, True)
        accum(t1, iidx, isum, False)
        accum(t2, iidx, isum, False)
        accum(p3, iidx, isum, False)
        accum(p3, ihi, isum, False)

        lanes = lax.iota(jnp.int32, 16)

        def dot_g(g, _):
            accv = jnp.zeros((16,), jnp.float32)
            for e in range(16):
                row = g * 16 + e
                sv = jnp.sum(usum[row, :] * isum[row, :]) * jnp.float32(0.0625)
                accv = jnp.where(lanes == e, sv, accv)
            outv[pl.ds(g * 16, 16)] = accv
            return 0

        lax.fori_loop(0, bpt // 16, dot_g, 0)
        pltpu.sync_copy(outv, out.at[pl.ds(b0, bpt)])

    return k


def kernel(users, items, user_emb_w, item_emb_w, graph_idx, graph_val):
    e0 = jnp.concatenate([user_emb_w, item_emb_w], axis=0)
    zpad = jnp.zeros((NP - N_NODES, EMB), jnp.float32)
    e0p = jnp.concatenate([e0, zpad], axis=0)
    nnz = graph_val.shape[0]
    cpt = -(-nnz // (NW * CH))
    cpt = -(-cpt // SSTAGE) * SSTAGE
    pad = NW * cpt * CH - nnz
    src = jnp.pad(graph_idx[1], (0, pad)).reshape(NW, cpt, CH)
    dst = jnp.pad(graph_idx[0], (0, pad)).reshape(NW, cpt, CH)
    val = jnp.pad(graph_val, (0, pad)).reshape(NW, cpt, CH)

    layer = _layer_kernel(cpt)
    tc_sum = _tc_pair_sum()

    def pair_sum(p):
        pr = p.reshape(2, _TCR, 128)
        return tc_sum(pr, pr).reshape(NP, EMB)

    p1 = layer(e0p, src, dst, val)
    t1 = pair_sum(p1)
    p2 = layer(t1, src, dst, val)
    t2 = pair_sum(p2)
    p3 = layer(t2, src, dst, val)
    scores = _score_kernel(int(users.shape[0]))(users, items, e0p, t1, t2, p3)
    return scores
